# Initial kernel scaffold; baseline (speedup 1.0000x reference)
import jax
import jax.numpy as jnp
from jax.experimental import pallas as pl


def kernel(hx, edgefeats, idxn, idxe, degs, Wf1, bf1, Wf2, bf2, W_ih, W_hh, b_ih, b_hh):
    raise NotImplementedError("write your pallas kernel here")


# speedup vs baseline: 4.6565x; 4.6565x over previous
import functools

import jax
import jax.numpy as jnp
from jax import lax
from jax.experimental import pallas as pl
from jax.experimental.pallas import tpu as pltpu
from jax.experimental.pallas import tpu_sc as plsc

_N = 10000
_DEG = 32
_E = _N * _DEG
_NC = 128
_DE = 16
_HID = 64

_NUM_WORKERS = 32
_EPW = _E // _NUM_WORKERS

_GA_BLK = 128
_GA_NFULL = _EPW // _GA_BLK
_GA_TAIL = _EPW - _GA_NFULL * _GA_BLK


def _gather_wg(weights, idxe):
    mesh = plsc.VectorSubcoreMesh(core_axis_name="c", subcore_axis_name="s")

    @functools.partial(
        pl.kernel, mesh=mesh,
        out_type=jax.ShapeDtypeStruct((_E, _NC), jnp.float32),
        scratch_types=[
            pltpu.VMEM((_GA_BLK,), jnp.int32),
            pltpu.VMEM((_GA_BLK, _NC), jnp.float32),
            pltpu.VMEM((_GA_TAIL,), jnp.int32),
            pltpu.VMEM((_GA_TAIL, _NC), jnp.float32),
            pltpu.SemaphoreType.DMA,
        ],
    )
    def k(w_hbm, idxe_hbm, out_hbm, idx_v, rows_v, idxt_v, rowst_v, sem):
        wid = lax.axis_index("s") * 2 + lax.axis_index("c")
        base = wid * _EPW

        def body(i, carry):
            off = base + i * _GA_BLK
            pltpu.sync_copy(idxe_hbm.at[pl.ds(off, _GA_BLK)], idx_v)
            pltpu.async_copy(w_hbm.at[idx_v], rows_v, sem).wait()
            pltpu.sync_copy(rows_v, out_hbm.at[pl.ds(off, _GA_BLK)])
            return carry

        lax.fori_loop(0, _GA_NFULL, body, 0)
        off = base + _GA_NFULL * _GA_BLK
        pltpu.sync_copy(idxe_hbm.at[pl.ds(off, _GA_TAIL)], idxt_v)
        pltpu.async_copy(w_hbm.at[idxt_v], rowst_v, sem).wait()
        pltpu.sync_copy(rowst_v, out_hbm.at[pl.ds(off, _GA_TAIL)])

    return k(weights, idxe)


_FB = 2560


def _filter_net(ef_g, Wf1, bf1, Wf2, bf2):
    def body(ef_ref, w1_ref, b1_ref, w2_ref, b2_ref, out_ref):
        h1 = jnp.dot(ef_ref[...], w1_ref[...],
                     preferred_element_type=jnp.float32) + b1_ref[...]
        h1 = jnp.maximum(h1, 0.0)
        out_ref[...] = jnp.dot(h1, w2_ref[...],
                               preferred_element_type=jnp.float32) + b2_ref[...]

    return pl.pallas_call(
        body,
        grid=(_E // _FB,),
        in_specs=[
            pl.BlockSpec((_FB, _DE), lambda i: (i, 0)),
            pl.BlockSpec((_DE, _HID), lambda i: (0, 0)),
            pl.BlockSpec((1, _HID), lambda i: (0, 0)),
            pl.BlockSpec((_HID, _NC), lambda i: (0, 0)),
            pl.BlockSpec((1, _NC), lambda i: (0, 0)),
        ],
        out_specs=pl.BlockSpec((_FB, _NC), lambda i: (i, 0)),
        out_shape=jax.ShapeDtypeStruct((_E, _NC), jnp.float32),
    )(ef_g, Wf1, bf1.reshape(1, _HID), Wf2, bf2.reshape(1, _NC))


_NBLK = 4
_EBLK = _NBLK * _DEG
_NBLOCKS = _N // _NBLK


def _gconv(h, wg, idxn):
    mesh = plsc.VectorSubcoreMesh(core_axis_name="c", subcore_axis_name="s")

    @functools.partial(
        pl.kernel, mesh=mesh,
        out_type=jax.ShapeDtypeStruct((_N, _NC), jnp.float32),
        scratch_types=[
            pltpu.VMEM((_EBLK,), jnp.int32),
            pltpu.VMEM((_EBLK, _NC), jnp.float32),
            pltpu.VMEM((_EBLK, _NC), jnp.float32),
            pltpu.VMEM((_NBLK, _NC), jnp.float32),
            pltpu.SemaphoreType.DMA,
            pltpu.SemaphoreType.DMA,
        ],
    )
    def k(h_hbm, wg_hbm, idxn_hbm, out_hbm, idx_v, rows_v, w_v, out_v,
          sem_g, sem_w):
        wid = lax.axis_index("s") * 2 + lax.axis_index("c")
        blo = (wid * _NBLOCKS) // _NUM_WORKERS
        bhi = ((wid + 1) * _NBLOCKS) // _NUM_WORKERS

        def body(b, carry):
            ebase = b * _EBLK
            pltpu.sync_copy(idxn_hbm.at[pl.ds(ebase, _EBLK)], idx_v)
            cp_w = pltpu.async_copy(wg_hbm.at[pl.ds(ebase, _EBLK)], w_v, sem_w)
            cp_g = pltpu.async_copy(h_hbm.at[idx_v], rows_v, sem_g)
            cp_w.wait()
            cp_g.wait()
            for nn in range(_NBLK):
                def ebody(j, accs):
                    e = nn * _DEG + j
                    return tuple(
                        accs[c] + rows_v[e, pl.ds(c * 16, 16)]
                        * w_v[e, pl.ds(c * 16, 16)]
                        for c in range(_NC // 16))

                accs = lax.fori_loop(
                    0, _DEG, ebody,
                    tuple(jnp.zeros((16,), jnp.float32)
                          for _ in range(_NC // 16)))
                for c in range(_NC // 16):
                    out_v[nn, pl.ds(c * 16, 16)] = accs[c]
            pltpu.sync_copy(out_v, out_hbm.at[pl.ds(b * _NBLK, _NBLK)])
            return carry

        lax.fori_loop(blo, bhi, body, 0)

    return k(h, wg, idxn)


_GB = 1000


def _gru(m, s, degs2, W_ih, W_hh, b_ih2, b_hh2):
    def body(m_ref, s_ref, d_ref, wih_ref, whh_ref, bih_ref, bhh_ref, out_ref):
        inv = 1.0 / jnp.maximum(d_ref[...].astype(jnp.float32), 1.0)
        x = m_ref[...] * inv
        gi = jnp.dot(x, wih_ref[...],
                     preferred_element_type=jnp.float32) + bih_ref[...]
        gh = jnp.dot(s_ref[...], whh_ref[...],
                     preferred_element_type=jnp.float32) + bhh_ref[...]
        ir, iz, inn = gi[:, :_NC], gi[:, _NC:2 * _NC], gi[:, 2 * _NC:]
        hr, hz, hn = gh[:, :_NC], gh[:, _NC:2 * _NC], gh[:, 2 * _NC:]
        r = jax.nn.sigmoid(ir + hr)
        z = jax.nn.sigmoid(iz + hz)
        nn_ = jnp.tanh(inn + r * hn)
        out_ref[...] = (1.0 - z) * nn_ + z * s_ref[...]

    return pl.pallas_call(
        body,
        grid=(_N // _GB,),
        in_specs=[
            pl.BlockSpec((_GB, _NC), lambda i: (i, 0)),
            pl.BlockSpec((_GB, _NC), lambda i: (i, 0)),
            pl.BlockSpec((_GB, 1), lambda i: (i, 0)),
            pl.BlockSpec((_NC, 3 * _NC), lambda i: (0, 0)),
            pl.BlockSpec((_NC, 3 * _NC), lambda i: (0, 0)),
            pl.BlockSpec((1, 3 * _NC), lambda i: (0, 0)),
            pl.BlockSpec((1, 3 * _NC), lambda i: (0, 0)),
        ],
        out_specs=pl.BlockSpec((_GB, _NC), lambda i: (i, 0)),
        out_shape=jax.ShapeDtypeStruct((_N, _NC), jnp.float32),
    )(m, s, degs2, W_ih, W_hh, b_ih2, b_hh2)


def kernel(hx, edgefeats, idxn, idxe, degs, Wf1, bf1, Wf2, bf2,
           W_ih, W_hh, b_ih, b_hh):
    weights = _filter_net(edgefeats, Wf1, bf1, Wf2, bf2)
    wg = _gather_wg(weights, idxe)
    degs2 = degs.reshape(_N, 1)
    bih2 = b_ih.reshape(1, 3 * _NC)
    bhh2 = b_hh.reshape(1, 3 * _NC)

    def g(s):
        m = _gconv(s, wg, idxn)
        return _gru(m, s, degs2, W_ih, W_hh, bih2, bhh2)

    hx1 = g(hx)
    hx2 = g(hx1)
    hx3 = g(hx2)
    sk1 = hx1 + hx3
    hx4 = g(sk1)
    hx5 = g(hx4)
    sk2 = hx3 + hx5
    hx6 = g(sk2)
    hx7 = g(hx6)
    sk3 = hx5 + hx7
    hx8 = g(sk3)
    hx9 = g(hx8)
    sk4 = hx7 + hx9
    hx10 = g(sk4)
    return jnp.concatenate(
        [hx, hx1, hx2, sk1, hx4, sk2, hx6, sk3, hx8, sk4, hx10], axis=1)


# speedup vs baseline: 8.4525x; 1.8152x over previous
import functools

import jax
import jax.numpy as jnp
from jax import lax
from jax.experimental import pallas as pl
from jax.experimental.pallas import tpu as pltpu
from jax.experimental.pallas import tpu_sc as plsc

_N = 10000
_DEG = 32
_E = _N * _DEG
_NC = 128
_DE = 16
_HID = 64

_NUM_WORKERS = 32
_EPW = _E // _NUM_WORKERS

_GA_BLK = 128
_GA_NBLOCKS = _E // _GA_BLK
_GA_MAXB = -(-_GA_NBLOCKS // _NUM_WORKERS)
_GA_NBUF = 4


def _gather_wg(weights, idxe2):
    mesh = plsc.VectorSubcoreMesh(core_axis_name="c", subcore_axis_name="s")

    @functools.partial(
        pl.kernel, mesh=mesh,
        out_type=jax.ShapeDtypeStruct((_E, _NC), jnp.float32),
        scratch_types=[
            pltpu.VMEM((_GA_MAXB, 1, _GA_BLK), jnp.int32),
        ] + [pltpu.VMEM((_GA_BLK, _NC), jnp.float32)] * _GA_NBUF
          + [pltpu.SemaphoreType.DMA] * (2 * _GA_NBUF),
    )
    def k(w_hbm, idxe_hbm, out_hbm, idx_v, *bufs_and_sems):
        rows = bufs_and_sems[:_GA_NBUF]
        gsems = bufs_and_sems[_GA_NBUF:2 * _GA_NBUF]
        ssems = bufs_and_sems[2 * _GA_NBUF:]
        wid = lax.axis_index("s") * 2 + lax.axis_index("c")
        blo = (wid * _GA_NBLOCKS) // _NUM_WORKERS
        bhi = ((wid + 1) * _GA_NBLOCKS) // _NUM_WORKERS
        nblk = bhi - blo
        pltpu.sync_copy(idxe_hbm.at[pl.ds(blo, _GA_MAXB)], idx_v)

        def fire(i, p):
            pltpu.async_copy(w_hbm.at[idx_v.at[i, 0]], rows[p], gsems[p])

        def wait_gather(i, p):
            pltpu.make_async_copy(w_hbm.at[idx_v.at[i, 0]], rows[p],
                                  gsems[p]).wait()

        def wait_store(p):
            pltpu.make_async_copy(rows[p], out_hbm.at[pl.ds(0, _GA_BLK)],
                                  ssems[p]).wait()

        fire(0, 0)

        @pl.when(nblk > 1)
        def _():
            fire(1, 1)

        def body(it, carry):
            for q in range(_GA_NBUF):
                i = it * _GA_NBUF + q
                p = q

                @pl.when(i < nblk)
                def _():
                    @pl.when(i >= 2)
                    def _():
                        wait_store((p + 2) % _GA_NBUF)

                    @pl.when(i + 2 < nblk)
                    def _():
                        fire(i + 2, (p + 2) % _GA_NBUF)

                    wait_gather(i, p)
                    pltpu.async_copy(
                        rows[p],
                        out_hbm.at[pl.ds((blo + i) * _GA_BLK, _GA_BLK)],
                        ssems[p])

            return carry

        lax.fori_loop(0, -(-_GA_MAXB // _GA_NBUF), body, 0)
        for p in range(_GA_NBUF):
            @pl.when(((nblk - 1) % _GA_NBUF == p)
                     | ((nblk - 2) % _GA_NBUF == p))
            def _():
                wait_store(p)

    return k(weights, idxe2)


_FB = 2560


def _filter_net(ef, Wf1, bf1, Wf2, bf2):
    def body(ef_ref, w1_ref, b1_ref, w2_ref, b2_ref, out_ref):
        h1 = jnp.dot(ef_ref[...], w1_ref[...],
                     preferred_element_type=jnp.float32) + b1_ref[...]
        h1 = jnp.maximum(h1, 0.0)
        out_ref[...] = jnp.dot(h1, w2_ref[...],
                               preferred_element_type=jnp.float32) + b2_ref[...]

    return pl.pallas_call(
        body,
        grid=(_E // _FB,),
        in_specs=[
            pl.BlockSpec((_FB, _DE), lambda i: (i, 0)),
            pl.BlockSpec((_DE, _HID), lambda i: (0, 0)),
            pl.BlockSpec((1, _HID), lambda i: (0, 0)),
            pl.BlockSpec((_HID, _NC), lambda i: (0, 0)),
            pl.BlockSpec((1, _NC), lambda i: (0, 0)),
        ],
        out_specs=pl.BlockSpec((_FB, _NC), lambda i: (i, 0)),
        out_shape=jax.ShapeDtypeStruct((_E, _NC), jnp.float32),
    )(ef, Wf1, bf1.reshape(1, _HID), Wf2, bf2.reshape(1, _NC))


_NBLK = 4
_EBLK = _NBLK * _DEG
_NBLOCKS = _N // _NBLK
_MAXB = -(-_NBLOCKS // _NUM_WORKERS)


def _gconv(h, wg, idxn2):
    mesh = plsc.VectorSubcoreMesh(core_axis_name="c", subcore_axis_name="s")

    @functools.partial(
        pl.kernel, mesh=mesh,
        out_type=jax.ShapeDtypeStruct((_N, _NC), jnp.float32),
        scratch_types=[
            pltpu.VMEM((_MAXB, 1, _EBLK), jnp.int32),
            pltpu.VMEM((_EBLK, _NC), jnp.float32),
            pltpu.VMEM((_EBLK, _NC), jnp.float32),
            pltpu.VMEM((_EBLK, _NC), jnp.float32),
            pltpu.VMEM((_EBLK, _NC), jnp.float32),
            pltpu.VMEM((_NBLK, _NC), jnp.float32),
            pltpu.VMEM((_NBLK, _NC), jnp.float32),
            pltpu.SemaphoreType.DMA,
            pltpu.SemaphoreType.DMA,
            pltpu.SemaphoreType.DMA,
            pltpu.SemaphoreType.DMA,
            pltpu.SemaphoreType.DMA,
            pltpu.SemaphoreType.DMA,
        ],
    )
    def k(h_hbm, wg_hbm, idxn_hbm, out_hbm, idx_v,
          r0_v, r1_v, w0_v, w1_v, o0_v, o1_v,
          sg0, sg1, sw0, sw1, so0, so1):
        wid = lax.axis_index("s") * 2 + lax.axis_index("c")
        blo = (wid * _NBLOCKS) // _NUM_WORKERS
        bhi = ((wid + 1) * _NBLOCKS) // _NUM_WORKERS
        nblk = bhi - blo
        pltpu.sync_copy(idxn_hbm.at[pl.ds(blo, _MAXB)], idx_v)
        rows = (r0_v, r1_v)
        wbuf = (w0_v, w1_v)
        obuf = (o0_v, o1_v)
        gsems = (sg0, sg1)
        wsems = (sw0, sw1)
        osems = (so0, so1)

        def fire(i, p):
            pltpu.async_copy(h_hbm.at[idx_v.at[i, 0]], rows[p], gsems[p])
            pltpu.async_copy(wg_hbm.at[pl.ds((blo + i) * _EBLK, _EBLK)],
                             wbuf[p], wsems[p])

        fire(0, 0)

        @pl.when(nblk > 1)
        def _():
            fire(1, 1)

        def body(it, carry):
            for p in range(2):
                i = it * 2 + p

                @pl.when(i < nblk)
                def _():
                    pltpu.make_async_copy(
                        h_hbm.at[idx_v.at[i, 0]], rows[p], gsems[p]).wait()
                    pltpu.make_async_copy(
                        wg_hbm.at[pl.ds(0, _EBLK)], wbuf[p], wsems[p]).wait()
                    @pl.when(i >= 2)
                    def _():
                        pltpu.make_async_copy(
                            obuf[p], out_hbm.at[pl.ds(0, _NBLK)],
                            osems[p]).wait()

                    for nn in range(_NBLK):
                        def ebody(j, accs):
                            e = nn * _DEG + j
                            return tuple(
                                accs[c] + rows[p][e, pl.ds(c * 16, 16)]
                                * wbuf[p][e, pl.ds(c * 16, 16)]
                                for c in range(_NC // 16))

                        accs = lax.fori_loop(
                            0, _DEG, ebody,
                            tuple(jnp.zeros((16,), jnp.float32)
                                  for _ in range(_NC // 16)),
                            unroll=2)
                        for c in range(_NC // 16):
                            obuf[p][nn, pl.ds(c * 16, 16)] = accs[c]

                    pltpu.async_copy(
                        obuf[p],
                        out_hbm.at[pl.ds((blo + i) * _NBLK, _NBLK)], osems[p])

                    @pl.when(i + 2 < nblk)
                    def _():
                        fire(i + 2, p)

            return carry

        lax.fori_loop(0, (nblk + 1) // 2, body, 0)
        pltpu.make_async_copy(obuf[0], out_hbm.at[pl.ds(0, _NBLK)],
                              osems[0]).wait()

        @pl.when(nblk > 1)
        def _():
            pltpu.make_async_copy(obuf[1], out_hbm.at[pl.ds(0, _NBLK)],
                                  osems[1]).wait()

    return k(h, wg, idxn2)


_GB = 1000


def _gru(m, s, degs2, W_ih, W_hh, b_ih2, b_hh2):
    def body(m_ref, s_ref, d_ref, wih_ref, whh_ref, bih_ref, bhh_ref, out_ref):
        inv = 1.0 / jnp.maximum(d_ref[...].astype(jnp.float32), 1.0)
        x = m_ref[...] * inv
        gi = jnp.dot(x, wih_ref[...],
                     preferred_element_type=jnp.float32) + bih_ref[...]
        gh = jnp.dot(s_ref[...], whh_ref[...],
                     preferred_element_type=jnp.float32) + bhh_ref[...]
        ir, iz, inn = gi[:, :_NC], gi[:, _NC:2 * _NC], gi[:, 2 * _NC:]
        hr, hz, hn = gh[:, :_NC], gh[:, _NC:2 * _NC], gh[:, 2 * _NC:]
        r = jax.nn.sigmoid(ir + hr)
        z = jax.nn.sigmoid(iz + hz)
        nn_ = jnp.tanh(inn + r * hn)
        out_ref[...] = (1.0 - z) * nn_ + z * s_ref[...]

    return pl.pallas_call(
        body,
        grid=(_N // _GB,),
        in_specs=[
            pl.BlockSpec((_GB, _NC), lambda i: (i, 0)),
            pl.BlockSpec((_GB, _NC), lambda i: (i, 0)),
            pl.BlockSpec((_GB, 1), lambda i: (i, 0)),
            pl.BlockSpec((_NC, 3 * _NC), lambda i: (0, 0)),
            pl.BlockSpec((_NC, 3 * _NC), lambda i: (0, 0)),
            pl.BlockSpec((1, 3 * _NC), lambda i: (0, 0)),
            pl.BlockSpec((1, 3 * _NC), lambda i: (0, 0)),
        ],
        out_specs=pl.BlockSpec((_GB, _NC), lambda i: (i, 0)),
        out_shape=jax.ShapeDtypeStruct((_N, _NC), jnp.float32),
    )(m, s, degs2, W_ih, W_hh, b_ih2, b_hh2)


def kernel(hx, edgefeats, idxn, idxe, degs, Wf1, bf1, Wf2, bf2,
           W_ih, W_hh, b_ih, b_hh):
    weights = _filter_net(edgefeats, Wf1, bf1, Wf2, bf2)
    wg = _gather_wg(weights, idxe.reshape(_GA_NBLOCKS, 1, _GA_BLK))
    idxn2 = idxn.reshape(_NBLOCKS, 1, _EBLK)
    degs2 = degs.reshape(_N, 1)
    bih2 = b_ih.reshape(1, 3 * _NC)
    bhh2 = b_hh.reshape(1, 3 * _NC)

    def g(s):
        m = _gconv(s, wg, idxn2)
        return _gru(m, s, degs2, W_ih, W_hh, bih2, bhh2)

    hx1 = g(hx)
    hx2 = g(hx1)
    hx3 = g(hx2)
    sk1 = hx1 + hx3
    hx4 = g(sk1)
    hx5 = g(hx4)
    sk2 = hx3 + hx5
    hx6 = g(sk2)
    hx7 = g(hx6)
    sk3 = hx5 + hx7
    hx8 = g(sk3)
    hx9 = g(hx8)
    sk4 = hx7 + hx9
    hx10 = g(sk4)
    return jnp.concatenate(
        [hx, hx1, hx2, sk1, hx4, sk2, hx6, sk3, hx8, sk4, hx10], axis=1)
